# baseline (speedup 1.0000x reference)
import jax
import jax.numpy as jnp
import numpy as np
from jax.experimental import pallas as pl

B = 8192
D_IN = 2048
NE = 5
OUT_DIMS = (4, 3, 2, 10, 1)
OUT_OFF = (0, 4, 7, 9, 19)
D_OUT = 20
BM = 1024

_NT = (((1,), (1,)), ((), ()))


def _gate_table_np():
    e5 = np.exp(np.float32(-5.0))
    s = np.float32(1.0) / (np.float32(1.0) + e5)
    c = e5 / (np.float32(1.0) + e5)
    t = np.zeros((10, 5), dtype=np.float32)
    for dt in range(10):
        m1 = dt in (4, 1, 2)
        m2 = dt in (0, 1, 5, 2)
        m4 = dt == 9
        gl = np.array([5.5, 0.5 + 10.0 * m1, 0.5 + 10.0 * m2, 0.5,
                       0.5 + 10.0 * m4], dtype=np.float32)
        idx = np.argsort(-gl, kind="stable")[:2]
        if gl[idx[0]] == gl[idx[1]]:
            w = np.array([0.5, 0.5], dtype=np.float32)
        else:
            w = np.array([s, c], dtype=np.float32)
        t[dt, idx[0]] = w[0]
        t[dt, idx[1]] = w[1]
    return t


_GATE_TABLE = _gate_table_np()
_GEXP = np.zeros((NE, D_OUT), dtype=np.float32)
for _i in range(NE):
    _GEXP[_i, OUT_OFF[_i]:OUT_OFF[_i] + OUT_DIMS[_i]] = 1.0


def _ln_lanes(h, g, b):
    m = jnp.mean(h, axis=-1, keepdims=True)
    d = h - m
    v = jnp.mean(d * d, axis=-1, keepdims=True)
    return d * jax.lax.rsqrt(v + 1e-5) * g + b


def _gelu(x):
    return x * 0.5 * (1.0 + jax.lax.erf(x * np.float32(0.7071067811865476)))


def _softplus(x):
    return jnp.maximum(x, 0.0) + jnp.log1p(jnp.exp(-jnp.abs(x)))


def _prep_kernel(w1_ref, *rest):
    b2_refs = rest[:NE]
    hb_refs = rest[NE:2 * NE]
    hw_refs = rest[2 * NE:3 * NE]
    w1bf_ref, tail_ref = rest[3 * NE], rest[3 * NE + 1]
    w1bf_ref[...] = w1_ref[...].astype(jnp.bfloat16)
    tail_ref[...] = jnp.zeros((22, D_OUT), jnp.float32)
    for i in range(NE):
        o0, od = OUT_OFF[i], OUT_DIMS[i]
        tail_ref[0:1, o0:o0 + od] = b2_refs[i][...][None, :]
        tail_ref[1:2, o0:o0 + od] = hb_refs[i][...][None, :]
        tail_ref[2 + o0:2 + o0 + od, o0:o0 + od] = hw_refs[i][...]


def _fused_kernel(dt_ref, sev_ref, loc_ref, x_ref,
                  w1_ref, b1_ref, g1_ref, be1_ref,
                  w2_ref, b2_ref,
                  emb_ref, mew_ref, meb_ref, meg_ref, mebe_ref,
                  gtab_ref, gexp_ref, tail_ref,
                  *rest):
    ex_refs = rest[:4 * NE]
    ew2_refs = rest[4 * NE:5 * NE]
    out_ref, gates_ref = rest[5 * NE], rest[5 * NE + 1]
    f32 = jnp.float32
    nt = lambda a, b: jax.lax.dot_general(a, b, _NT, preferred_element_type=f32)

    h = nt(x_ref[...].astype(jnp.bfloat16), w1_ref[...]) + b1_ref[...][None, :]
    h = _gelu(_ln_lanes(h, g1_ref[...][None, :], be1_ref[...][None, :]))
    enc = nt(h, w2_ref[...]) + b2_ref[...][None, :]

    dt = dt_ref[...]
    lane10 = jax.lax.broadcasted_iota(jnp.int32, (BM, 10), 1)
    oh = (dt == lane10).astype(f32)
    temb = jnp.dot(oh, emb_ref[...], preferred_element_type=f32)
    meta = jnp.concatenate([temb, sev_ref[...], loc_ref[...]], axis=-1)
    mp = nt(meta, mew_ref[...]) + meb_ref[...][None, :]
    meta_enc = _gelu(_ln_lanes(mp, meg_ref[...][None, :], mebe_ref[...][None, :]))

    ex_in = jnp.concatenate([enc, meta_enc], axis=-1)
    ex_bf = ex_in.astype(jnp.bfloat16)
    gates = jnp.dot(oh, gtab_ref[...], preferred_element_type=f32)
    outs = []
    for i in range(NE):
        eW1, eb1, eg, ebe = ex_refs[4 * i:4 * i + 4]
        hi = nt(ex_bf, eW1[...].astype(jnp.bfloat16)) + eb1[...][None, :]
        hi = _gelu(_ln_lanes(hi, eg[...][None, :], ebe[...][None, :]))
        outs.append(nt(hi, ew2_refs[i][...]))
    o = jnp.concatenate(outs, axis=-1) + tail_ref[0:1, :]

    col = jax.lax.broadcasted_iota(jnp.int32, (BM, D_OUT), 1)
    m_sm0 = col < 4
    m_sm3 = (col >= 9) & (col < 19)
    m_sig = col >= 19

    def _masked_softmax(mask):
        xm = jnp.where(mask, o, -1e30)
        mx = jnp.max(xm, axis=-1, keepdims=True)
        e = jnp.exp(xm - mx)
        return e / jnp.sum(e, axis=-1, keepdims=True)

    o_act = jnp.where(m_sm0, _masked_softmax(m_sm0),
                      jnp.where(m_sm3, _masked_softmax(m_sm3),
                                jnp.where(m_sig, jax.nn.sigmoid(o),
                                          _softplus(o))))
    o2 = nt(o_act, tail_ref[2:22, :]) + tail_ref[1:2, :]
    gcols = jnp.dot(gates, gexp_ref[...], preferred_element_type=f32)
    out_ref[...] = o2 * gcols
    gates_ref[...] = gates


@jax.jit
def _run(x, dt2d, severity, location, params):
    p = params
    ex = p['experts']

    def row2(v):
        return v.reshape(1, v.shape[0])

    prep_in = ([p['enc_W1']] + [e['b2'] for e in ex]
               + [e['hb'] for e in ex] + [e['hW'] for e in ex])
    w1bf, tail = pl.pallas_call(
        _prep_kernel,
        out_shape=[jax.ShapeDtypeStruct((128, D_IN), jnp.bfloat16),
                   jax.ShapeDtypeStruct((22, D_OUT), jnp.float32)],
    )(*prep_in)

    consts = [w1bf, p['enc_b1'], p['enc_g1'], p['enc_be1'],
              p['enc_W2'], p['enc_b2'],
              p['emb'], p['meW'], p['meb'], p['meg'],
              p['mebeta'],
              jnp.asarray(_GATE_TABLE), jnp.asarray(_GEXP), tail]
    for e in ex:
        consts += [e['W1'], e['b1'], e['g'], e['beta']]
    consts += [e['W2'] for e in ex]

    grid = (B // BM,)
    bs_row = lambda n: pl.BlockSpec((BM, n), lambda i: (i, 0))
    bs_full = lambda a: pl.BlockSpec(a.shape, lambda i: (0,) * a.ndim)
    out, gates = pl.pallas_call(
        _fused_kernel,
        grid=grid,
        in_specs=[bs_row(1), bs_row(4), bs_row(2), bs_row(D_IN)]
                 + [bs_full(a) for a in consts],
        out_specs=[bs_row(D_OUT), bs_row(NE)],
        out_shape=[jax.ShapeDtypeStruct((B, D_OUT), jnp.float32),
                   jax.ShapeDtypeStruct((B, NE), jnp.float32)],
    )(dt2d, severity, location, x, *consts)
    return out, gates


def kernel(x, disaster_type, severity, location, params):
    dt2d = disaster_type.reshape(B, 1)
    return _run(x, dt2d, severity, location, params)


# speedup vs baseline: 1.0163x; 1.0163x over previous
import jax
import jax.numpy as jnp
import numpy as np
from jax.experimental import pallas as pl
from jax.experimental.pallas import tpu as pltpu

B = 8192
D_IN = 2048
NE = 5
OUT_DIMS = (4, 3, 2, 10, 1)
OUT_OFF = (0, 4, 7, 9, 19)
D_OUT = 20
BM = 1024
NS = B // BM

_NT = (((1,), (1,)), ((), ()))


def _gate_table_np():
    e5 = np.exp(np.float32(-5.0))
    s = np.float32(1.0) / (np.float32(1.0) + e5)
    c = e5 / (np.float32(1.0) + e5)
    t = np.zeros((10, 5), dtype=np.float32)
    for dt in range(10):
        m1 = dt in (4, 1, 2)
        m2 = dt in (0, 1, 5, 2)
        m4 = dt == 9
        gl = np.array([5.5, 0.5 + 10.0 * m1, 0.5 + 10.0 * m2, 0.5,
                       0.5 + 10.0 * m4], dtype=np.float32)
        idx = np.argsort(-gl, kind="stable")[:2]
        if gl[idx[0]] == gl[idx[1]]:
            w = np.array([0.5, 0.5], dtype=np.float32)
        else:
            w = np.array([s, c], dtype=np.float32)
        t[dt, idx[0]] = w[0]
        t[dt, idx[1]] = w[1]
    return t


_GATE_TABLE = _gate_table_np()
_GEXP = np.zeros((NE, D_OUT), dtype=np.float32)
for _i in range(NE):
    _GEXP[_i, OUT_OFF[_i]:OUT_OFF[_i] + OUT_DIMS[_i]] = 1.0


def _ln_lanes(h, g, b):
    m = jnp.mean(h, axis=-1, keepdims=True)
    v = jnp.mean(h * h, axis=-1, keepdims=True) - m * m
    return (h - m) * jax.lax.rsqrt(v + 1e-5) * g + b


def _gelu(x):
    return x * 0.5 * (1.0 + jax.lax.erf(x * np.float32(0.7071067811865476)))


def _softplus(x):
    return jnp.maximum(x, 0.0) + jnp.log1p(jnp.exp(-jnp.abs(x)))


def _fused_kernel(dt_ref, sev_ref, loc_ref, x_ref,
                  w1_ref, b1_ref, g1_ref, be1_ref,
                  w2_ref, b2_ref,
                  emb_ref, mew_ref, meb_ref, meg_ref, mebe_ref,
                  gtab_ref, gexp_ref,
                  ew1_0, ew1_1, ew1_2, ew1_3, ew1_4,
                  eb1_0, eb1_1, eb1_2, eb1_3, eb1_4,
                  eg_0, eg_1, eg_2, eg_3, eg_4,
                  ebe_0, ebe_1, ebe_2, ebe_3, ebe_4,
                  ew2_0, ew2_1, ew2_2, ew2_3, ew2_4,
                  b2_0, b2_1, b2_2, b2_3, b2_4,
                  hb_0, hb_1, hb_2, hb_3, hb_4,
                  hw_0, hw_1, hw_2, hw_3, hw_4,
                  out_ref, gates_ref,
                  ew1c_s, tail_s):
    ew1_refs = (ew1_0, ew1_1, ew1_2, ew1_3, ew1_4)
    eb1_refs = (eb1_0, eb1_1, eb1_2, eb1_3, eb1_4)
    eg_refs = (eg_0, eg_1, eg_2, eg_3, eg_4)
    ebe_refs = (ebe_0, ebe_1, ebe_2, ebe_3, ebe_4)
    ew2_refs = (ew2_0, ew2_1, ew2_2, ew2_3, ew2_4)
    b2_refs = (b2_0, b2_1, b2_2, b2_3, b2_4)
    hb_refs = (hb_0, hb_1, hb_2, hb_3, hb_4)
    hw_refs = (hw_0, hw_1, hw_2, hw_3, hw_4)
    f32 = jnp.float32
    nt = lambda a, b: jax.lax.dot_general(a, b, _NT, preferred_element_type=f32)
    pid = pl.program_id(0)

    @pl.when(pid == 0)
    def _prep():
        tail_s[...] = jnp.zeros((22, D_OUT), f32)
        for i in range(NE):
            ew1c_s[128 * i:128 * (i + 1), :] = ew1_refs[i][...].astype(jnp.bfloat16)
            o0, od = OUT_OFF[i], OUT_DIMS[i]
            tail_s[0:1, o0:o0 + od] = b2_refs[i][...][None, :]
            tail_s[1:2, o0:o0 + od] = hb_refs[i][...][None, :]
            tail_s[2 + o0:2 + o0 + od, o0:o0 + od] = hw_refs[i][...]

    h = nt(x_ref[...], w1_ref[...]) + b1_ref[...][None, :]
    h = _gelu(_ln_lanes(h, g1_ref[...][None, :], be1_ref[...][None, :]))
    enc = nt(h, w2_ref[...]) + b2_ref[...][None, :]

    dt = dt_ref[...]
    lane10 = jax.lax.broadcasted_iota(jnp.int32, (BM, 10), 1)
    oh = (dt == lane10).astype(f32)
    temb = jnp.dot(oh, emb_ref[...], preferred_element_type=f32)
    meta = jnp.concatenate([temb, sev_ref[...], loc_ref[...]], axis=-1)
    mp = nt(meta, mew_ref[...]) + meb_ref[...][None, :]
    meta_enc = _gelu(_ln_lanes(mp, meg_ref[...][None, :], mebe_ref[...][None, :]))

    ex_in = jnp.concatenate([enc, meta_enc], axis=-1)
    ex_bf = ex_in.astype(jnp.bfloat16)
    gates = jnp.dot(oh, gtab_ref[...], preferred_element_type=f32)
    h5 = nt(ex_bf, ew1c_s[...])
    outs = []
    for i in range(NE):
        sl = slice(128 * i, 128 * (i + 1))
        hi = h5[:, sl] + eb1_refs[i][...][None, :]
        hi = _gelu(_ln_lanes(hi, eg_refs[i][...][None, :],
                             ebe_refs[i][...][None, :]))
        outs.append(nt(hi, ew2_refs[i][...]))
    o = jnp.concatenate(outs, axis=-1) + tail_s[0:1, :]

    col = jax.lax.broadcasted_iota(jnp.int32, (BM, D_OUT), 1)
    m_sm0 = col < 4
    m_sm3 = (col >= 9) & (col < 19)
    m_sig = col >= 19

    def _masked_softmax(mask):
        xm = jnp.where(mask, o, -1e30)
        mx = jnp.max(xm, axis=-1, keepdims=True)
        e = jnp.exp(xm - mx)
        return e / jnp.sum(e, axis=-1, keepdims=True)

    o_act = jnp.where(m_sm0, _masked_softmax(m_sm0),
                      jnp.where(m_sm3, _masked_softmax(m_sm3),
                                jnp.where(m_sig, jax.nn.sigmoid(o),
                                          _softplus(o))))
    o2 = nt(o_act, tail_s[2:22, :]) + tail_s[1:2, :]
    gcols = jnp.dot(gates, gexp_ref[...], preferred_element_type=f32)
    out_ref[...] = o2 * gcols
    gates_ref[...] = gates


@jax.jit
def _run(x, dt2d, severity, location, params):
    p = params
    ex = p['experts']

    consts = ([p['enc_W1'], p['enc_b1'], p['enc_g1'], p['enc_be1'],
               p['enc_W2'], p['enc_b2'],
               p['emb'], p['meW'], p['meb'], p['meg'], p['mebeta'],
               jnp.asarray(_GATE_TABLE), jnp.asarray(_GEXP)]
              + [e['W1'] for e in ex]
              + [e['b1'] for e in ex]
              + [e['g'] for e in ex]
              + [e['beta'] for e in ex]
              + [e['W2'] for e in ex]
              + [e['b2'] for e in ex]
              + [e['hb'] for e in ex]
              + [e['hW'] for e in ex])

    grid = (NS,)
    bs_row = lambda n: pl.BlockSpec((BM, n), lambda i: (i, 0))
    bs_full = lambda a: pl.BlockSpec(a.shape, lambda i: (0,) * a.ndim)
    out, gates = pl.pallas_call(
        _fused_kernel,
        grid=grid,
        in_specs=[bs_row(1), bs_row(4), bs_row(2), bs_row(D_IN)]
                 + [bs_full(a) for a in consts],
        out_specs=[bs_row(D_OUT), bs_row(NE)],
        out_shape=[jax.ShapeDtypeStruct((B, D_OUT), jnp.float32),
                   jax.ShapeDtypeStruct((B, NE), jnp.float32)],
        scratch_shapes=[pltpu.VMEM((NE * 128, 128), jnp.bfloat16),
                        pltpu.VMEM((22, D_OUT), jnp.float32)],
    )(dt2d, severity, location, x, *consts)
    return out, gates


def kernel(x, disaster_type, severity, location, params):
    dt2d = disaster_type.reshape(B, 1)
    return _run(x, dt2d, severity, location, params)


# speedup vs baseline: 1.0496x; 1.0328x over previous
import jax
import jax.numpy as jnp
import numpy as np
from jax.experimental import pallas as pl
from jax.experimental.pallas import tpu as pltpu

B = 8192
D_IN = 2048
NE = 5
OUT_DIMS = (4, 3, 2, 10, 1)
OUT_OFF = (0, 4, 7, 9, 19)
D_OUT = 20
BM = 1024
NS = B // BM

_NT = (((1,), (1,)), ((), ()))


def _gate_table_np():
    e5 = np.exp(np.float32(-5.0))
    s = np.float32(1.0) / (np.float32(1.0) + e5)
    c = e5 / (np.float32(1.0) + e5)
    t = np.zeros((10, 5), dtype=np.float32)
    for dt in range(10):
        m1 = dt in (4, 1, 2)
        m2 = dt in (0, 1, 5, 2)
        m4 = dt == 9
        gl = np.array([5.5, 0.5 + 10.0 * m1, 0.5 + 10.0 * m2, 0.5,
                       0.5 + 10.0 * m4], dtype=np.float32)
        idx = np.argsort(-gl, kind="stable")[:2]
        if gl[idx[0]] == gl[idx[1]]:
            w = np.array([0.5, 0.5], dtype=np.float32)
        else:
            w = np.array([s, c], dtype=np.float32)
        t[dt, idx[0]] = w[0]
        t[dt, idx[1]] = w[1]
    return t


_GATE_TABLE = _gate_table_np()
_GEXP = np.zeros((NE, D_OUT), dtype=np.float32)
for _i in range(NE):
    _GEXP[_i, OUT_OFF[_i]:OUT_OFF[_i] + OUT_DIMS[_i]] = 1.0


def _ln_lanes(h, g, b):
    m = jnp.mean(h, axis=-1, keepdims=True)
    v = jnp.mean(h * h, axis=-1, keepdims=True) - m * m
    return (h - m) * jax.lax.rsqrt(v + 1e-5) * g + b


def _gelu(x):
    return x * 0.5 * (1.0 + jax.lax.erf(x * np.float32(0.7071067811865476)))


def _softplus(x):
    return jnp.maximum(x, 0.0) + jnp.log1p(jnp.exp(-jnp.abs(x)))


def _fused_kernel(dt_ref, sev_ref, loc_ref, x_ref,
                  w1_ref, b1_ref, g1_ref, be1_ref,
                  w2_ref, b2_ref,
                  emb_ref, mew_ref, meb_ref, meg_ref, mebe_ref,
                  gtab_ref, gexp_ref,
                  ew1_0, ew1_1, ew1_2, ew1_3, ew1_4,
                  eb1_0, eb1_1, eb1_2, eb1_3, eb1_4,
                  eg_0, eg_1, eg_2, eg_3, eg_4,
                  ebe_0, ebe_1, ebe_2, ebe_3, ebe_4,
                  ew2_0, ew2_1, ew2_2, ew2_3, ew2_4,
                  b2_0, b2_1, b2_2, b2_3, b2_4,
                  hb_0, hb_1, hb_2, hb_3, hb_4,
                  hw_0, hw_1, hw_2, hw_3, hw_4,
                  out_ref, gates_ref,
                  ew1c_s, tail_s):
    ew1_refs = (ew1_0, ew1_1, ew1_2, ew1_3, ew1_4)
    eb1_refs = (eb1_0, eb1_1, eb1_2, eb1_3, eb1_4)
    eg_refs = (eg_0, eg_1, eg_2, eg_3, eg_4)
    ebe_refs = (ebe_0, ebe_1, ebe_2, ebe_3, ebe_4)
    ew2_refs = (ew2_0, ew2_1, ew2_2, ew2_3, ew2_4)
    b2_refs = (b2_0, b2_1, b2_2, b2_3, b2_4)
    hb_refs = (hb_0, hb_1, hb_2, hb_3, hb_4)
    hw_refs = (hw_0, hw_1, hw_2, hw_3, hw_4)
    f32 = jnp.float32
    nt = lambda a, b: jax.lax.dot_general(a, b, _NT, preferred_element_type=f32)
    pid = pl.program_id(0)

    @pl.when(pid == 0)
    def _prep():
        tail_s[...] = jnp.zeros((22, D_OUT), f32)
        for i in range(NE):
            ew1c_s[128 * i:128 * (i + 1), :] = ew1_refs[i][...].astype(jnp.bfloat16)
            o0, od = OUT_OFF[i], OUT_DIMS[i]
            tail_s[0:1, o0:o0 + od] = b2_refs[i][...][None, :]
            tail_s[1:2, o0:o0 + od] = hb_refs[i][...][None, :]
            tail_s[2 + o0:2 + o0 + od, o0:o0 + od] = hw_refs[i][...]

    h = nt(x_ref[...], w1_ref[...]) + b1_ref[...][None, :]
    h = _gelu(_ln_lanes(h, g1_ref[...][None, :], be1_ref[...][None, :]))
    enc = nt(h, w2_ref[...]) + b2_ref[...][None, :]

    dt = dt_ref[...][:, None]
    lane10 = jax.lax.broadcasted_iota(jnp.int32, (BM, 10), 1)
    oh = (dt == lane10).astype(f32)
    temb = jnp.dot(oh, emb_ref[...], preferred_element_type=f32)
    meta = jnp.concatenate([temb, sev_ref[...], loc_ref[...]], axis=-1)
    mp = nt(meta, mew_ref[...]) + meb_ref[...][None, :]
    meta_enc = _gelu(_ln_lanes(mp, meg_ref[...][None, :], mebe_ref[...][None, :]))

    ex_in = jnp.concatenate([enc, meta_enc], axis=-1)
    ex_bf = ex_in.astype(jnp.bfloat16)
    gates = jnp.dot(oh, gtab_ref[...], preferred_element_type=f32)
    h5 = nt(ex_bf, ew1c_s[...])
    outs = []
    for i in range(NE):
        sl = slice(128 * i, 128 * (i + 1))
        hi = h5[:, sl] + eb1_refs[i][...][None, :]
        hi = _gelu(_ln_lanes(hi, eg_refs[i][...][None, :],
                             ebe_refs[i][...][None, :]))
        outs.append(nt(hi, ew2_refs[i][...]))
    o = jnp.concatenate(outs, axis=-1) + tail_s[0:1, :]

    col = jax.lax.broadcasted_iota(jnp.int32, (BM, D_OUT), 1)
    m_sm0 = col < 4
    m_sm3 = (col >= 9) & (col < 19)
    m_sig = col >= 19

    def _masked_softmax(mask):
        xm = jnp.where(mask, o, -1e30)
        mx = jnp.max(xm, axis=-1, keepdims=True)
        e = jnp.exp(xm - mx)
        return e / jnp.sum(e, axis=-1, keepdims=True)

    o_act = jnp.where(m_sm0, _masked_softmax(m_sm0),
                      jnp.where(m_sm3, _masked_softmax(m_sm3),
                                jnp.where(m_sig, jax.nn.sigmoid(o),
                                          _softplus(o))))
    o2 = nt(o_act, tail_s[2:22, :]) + tail_s[1:2, :]
    gcols = jnp.dot(gates, gexp_ref[...], preferred_element_type=f32)
    out_ref[...] = o2 * gcols
    gates_ref[...] = gates


@jax.jit
def _run(x, dt2d, severity, location, params):
    p = params
    ex = p['experts']

    consts = ([p['enc_W1'], p['enc_b1'], p['enc_g1'], p['enc_be1'],
               p['enc_W2'], p['enc_b2'],
               p['emb'], p['meW'], p['meb'], p['meg'], p['mebeta'],
               jnp.asarray(_GATE_TABLE), jnp.asarray(_GEXP)]
              + [e['W1'] for e in ex]
              + [e['b1'] for e in ex]
              + [e['g'] for e in ex]
              + [e['beta'] for e in ex]
              + [e['W2'] for e in ex]
              + [e['b2'] for e in ex]
              + [e['hb'] for e in ex]
              + [e['hW'] for e in ex])

    grid = (NS,)
    bs_row = lambda n: pl.BlockSpec((BM, n), lambda i: (i, 0))
    bs_full = lambda a: pl.BlockSpec(a.shape, lambda i: (0,) * a.ndim)
    out, gates = pl.pallas_call(
        _fused_kernel,
        grid=grid,
        in_specs=[pl.BlockSpec((BM,), lambda i: (i,)),
                  bs_row(4), bs_row(2), bs_row(D_IN)]
                 + [bs_full(a) for a in consts],
        out_specs=[bs_row(D_OUT), bs_row(NE)],
        out_shape=[jax.ShapeDtypeStruct((B, D_OUT), jnp.float32),
                   jax.ShapeDtypeStruct((B, NE), jnp.float32)],
        scratch_shapes=[pltpu.VMEM((NE * 128, 128), jnp.bfloat16),
                        pltpu.VMEM((22, D_OUT), jnp.float32)],
    )(dt2d, severity, location, x, *consts)
    return out, gates


def kernel(x, disaster_type, severity, location, params):
    return _run(x, disaster_type, severity, location, params)
